# baseline (speedup 1.0000x reference)
import jax
import jax.numpy as jnp
from jax import lax
from jax.experimental import pallas as pl
from jax.experimental.pallas import tpu as pltpu
from jax.experimental.pallas import tpu_sc as plsc

_K = 1024
_D = 32
_N = 65536
_B = 4
_R = _B * 1024
_GRID = _N // _R
_CC = 6.0
_EPS = 1e-05

_NW = 32
_RPW = _N // _NW
_CHUNK = 128
_NCHUNK = _RPW // _CHUNK


def _tc_body(x_ref, e_ref, idx_ref, idxsc_ref, loss_ref, ssd_ref):
    i = pl.program_id(0)

    @pl.when(i == 0)
    def _init():
        ssd_ref[0] = 0.0

    x = x_ref[...].reshape(_R, _D)
    e = e_ref[...]
    score = jnp.dot(x, e, preferred_element_type=jnp.float32)
    x2 = jnp.sum(x * x, axis=1, keepdims=True)
    e2 = jnp.sum(e * e, axis=0, keepdims=True)
    dist = (x2 + e2) - 2.0 * score
    idx = jnp.argmin(dist, axis=1).astype(jnp.int32)
    idx_ref[...] = idx.reshape(_B, 32, 32)
    idxsc_ref[...] = idx.reshape(_R // 2048, 16, 128)
    one_hot = (lax.broadcasted_iota(jnp.int32, (_R, _K), 1)
               == idx[:, None]).astype(jnp.float32)
    q = lax.dot_general(one_hot, e, (((1,), (1,)), ((), ())),
                        preferred_element_type=jnp.float32)
    d = q - x
    ssd_ref[0] += jnp.sum(d * d)

    @pl.when(i == _GRID - 1)
    def _fin():
        loss_ref[...] = jnp.full((1, 1), (_CC / (_N * _D)) * ssd_ref[0],
                                 jnp.float32)


def _sc_body(table_hbm, idxsc_hbm, q_hbm, cnt_hbm, idx_v, rows_v, cnt_v, sem):
    cid = lax.axis_index("c")
    sid = lax.axis_index("s")
    w = sid * 2 + cid
    pltpu.sync_copy(idxsc_hbm.at[w], idx_v)
    copies = []
    for j in range(_NCHUNK):
        copies.append(pltpu.async_copy(
            table_hbm.at[idx_v.at[j]],
            rows_v.at[pl.ds(j * _CHUNK, _CHUNK)], sem))
    for c in copies:
        c.wait()
    pltpu.sync_copy(rows_v, q_hbm.at[pl.ds(w * _RPW, _RPW)])
    for r in range(8):
        for c in range(8):
            cnt_v[r, pl.ds(c * 16, 16)] = jnp.zeros((16,), jnp.float32)
    ones = jnp.ones((16,), jnp.float32)
    for j in range(16):
        for g in range(8):
            iv = idx_v[j, pl.ds(g * 16, 16)]
            plsc.addupdate_scatter(
                cnt_v, [lax.shift_right_logical(iv, 7),
                        jnp.bitwise_and(iv, 127)], ones)
    pltpu.sync_copy(cnt_v, cnt_hbm.at[w])


def _perp_body(cnt_ref, perp_ref):
    c = jnp.sum(cnt_ref[...], axis=0)
    p = c * (1.0 / _N)
    ent = jnp.sum(p * jnp.log(p + _EPS), axis=(0, 1), keepdims=True)
    perp_ref[...] = jnp.exp(-ent)


def kernel(inputs, embeddings):
    idx, idx_sc, loss = pl.pallas_call(
        _tc_body,
        grid=(_GRID,),
        in_specs=[
            pl.BlockSpec((_B, 32, 32, _D), lambda i: (i, 0, 0, 0)),
            pl.BlockSpec((_D, _K), lambda i: (0, 0)),
        ],
        out_specs=[
            pl.BlockSpec((_B, 32, 32), lambda i: (i, 0, 0)),
            pl.BlockSpec((_R // 2048, 16, 128), lambda i: (i, 0, 0)),
            pl.BlockSpec((1, 1), lambda i: (0, 0)),
        ],
        out_shape=[
            jax.ShapeDtypeStruct((64, 32, 32), jnp.int32),
            jax.ShapeDtypeStruct((_NW, 16, 128), jnp.int32),
            jax.ShapeDtypeStruct((1, 1), jnp.float32),
        ],
        scratch_shapes=[
            pltpu.SMEM((1,), jnp.float32),
        ],
    )(inputs, embeddings)

    table = embeddings.T
    mesh = plsc.VectorSubcoreMesh(core_axis_name="c", subcore_axis_name="s")
    q_flat, cnt_parts = pl.kernel(
        _sc_body,
        mesh=mesh,
        compiler_params=pltpu.CompilerParams(use_tc_tiling_on_sc=False,
                                             needs_layout_passes=False),
        out_type=[
            jax.ShapeDtypeStruct((_N, _D), jnp.float32),
            jax.ShapeDtypeStruct((_NW, 8, 128), jnp.float32),
        ],
        scratch_types=[
            pltpu.VMEM((16, 128), jnp.int32),
            pltpu.VMEM((_RPW, _D), jnp.float32),
            pltpu.VMEM((8, 128), jnp.float32),
            pltpu.SemaphoreType.DMA,
        ],
    )(table, idx_sc)

    perp = pl.pallas_call(
        _perp_body,
        out_shape=jax.ShapeDtypeStruct((1, 1), jnp.float32),
    )(cnt_parts)

    q = q_flat.reshape(inputs.shape)
    return (loss.reshape(()), q, idx, perp.reshape(()))


# speedup vs baseline: 1.1594x; 1.1594x over previous
import jax
import jax.numpy as jnp
from jax import lax
from jax.experimental import pallas as pl
from jax.experimental.pallas import tpu as pltpu
from jax.experimental.pallas import tpu_sc as plsc

_K = 1024
_D = 32
_N = 65536
_B = 4
_R = _B * 1024
_GRID = _N // _R
_CC = 6.0
_EPS = 1e-05

_NW = 32
_RPW = _N // _NW


def _tc_body(x_ref, e_ref, q_ref, idx_ref, idxsc_ref, loss_ref, ssd_ref):
    i = pl.program_id(0)

    @pl.when(i == 0)
    def _init():
        ssd_ref[0] = 0.0

    x = x_ref[...].reshape(_R, _D)
    e = e_ref[...]
    score = jnp.dot(x, e, preferred_element_type=jnp.float32)
    x2 = jnp.sum(x * x, axis=1, keepdims=True)
    e2 = jnp.sum(e * e, axis=0, keepdims=True)
    dist = (x2 + e2) - 2.0 * score
    idx = jnp.argmin(dist, axis=1).astype(jnp.int32)
    idx_ref[...] = idx.reshape(_B, 32, 32)
    idxsc_ref[...] = idx.reshape(_R // _RPW, 16, 128)
    one_hot = (lax.broadcasted_iota(jnp.int32, (_R, _K), 1)
               == idx[:, None]).astype(jnp.float32)
    q = lax.dot_general(one_hot, e, (((1,), (1,)), ((), ())),
                        preferred_element_type=jnp.float32)
    q_ref[...] = q.reshape(_B, 32, 32, _D)
    d = q - x
    ssd_ref[0] += jnp.sum(d * d)

    @pl.when(i == _GRID - 1)
    def _fin():
        loss_ref[...] = jnp.full((1, 1), (_CC / (_N * _D)) * ssd_ref[0],
                                 jnp.float32)


def _sc_body(idxsc_hbm, cnt_hbm, idx_v, cnt_v):
    cid = lax.axis_index("c")
    sid = lax.axis_index("s")
    w = sid * 2 + cid
    pltpu.sync_copy(idxsc_hbm.at[w], idx_v)
    for r in range(8):
        for c in range(8):
            cnt_v[r, pl.ds(c * 16, 16)] = jnp.zeros((16,), jnp.float32)
    ones = jnp.ones((16,), jnp.float32)
    for j in range(16):
        for g in range(8):
            iv = idx_v[j, pl.ds(g * 16, 16)]
            plsc.addupdate_scatter(
                cnt_v, [lax.shift_right_logical(iv, 7),
                        jnp.bitwise_and(iv, 127)], ones)
    pltpu.sync_copy(cnt_v, cnt_hbm.at[w])


def _perp_body(cnt_ref, perp_ref):
    c = jnp.sum(cnt_ref[...], axis=0)
    p = c * (1.0 / _N)
    ent = jnp.sum(p * jnp.log(p + _EPS), axis=(0, 1), keepdims=True)
    perp_ref[...] = jnp.exp(-ent)


def kernel(inputs, embeddings):
    q, idx, idx_sc, loss = pl.pallas_call(
        _tc_body,
        grid=(_GRID,),
        in_specs=[
            pl.BlockSpec((_B, 32, 32, _D), lambda i: (i, 0, 0, 0)),
            pl.BlockSpec((_D, _K), lambda i: (0, 0)),
        ],
        out_specs=[
            pl.BlockSpec((_B, 32, 32, _D), lambda i: (i, 0, 0, 0)),
            pl.BlockSpec((_B, 32, 32), lambda i: (i, 0, 0)),
            pl.BlockSpec((_R // _RPW, 16, 128), lambda i: (i, 0, 0)),
            pl.BlockSpec((1, 1), lambda i: (0, 0)),
        ],
        out_shape=[
            jax.ShapeDtypeStruct((64, 32, 32, _D), jnp.float32),
            jax.ShapeDtypeStruct((64, 32, 32), jnp.int32),
            jax.ShapeDtypeStruct((_NW, 16, 128), jnp.int32),
            jax.ShapeDtypeStruct((1, 1), jnp.float32),
        ],
        scratch_shapes=[
            pltpu.SMEM((1,), jnp.float32),
        ],
    )(inputs, embeddings)

    mesh = plsc.VectorSubcoreMesh(core_axis_name="c", subcore_axis_name="s")
    cnt_parts = pl.kernel(
        _sc_body,
        mesh=mesh,
        compiler_params=pltpu.CompilerParams(use_tc_tiling_on_sc=False,
                                             needs_layout_passes=False),
        out_type=jax.ShapeDtypeStruct((_NW, 8, 128), jnp.float32),
        scratch_types=[
            pltpu.VMEM((16, 128), jnp.int32),
            pltpu.VMEM((8, 128), jnp.float32),
        ],
    )(idx_sc)

    perp = pl.pallas_call(
        _perp_body,
        out_shape=jax.ShapeDtypeStruct((1, 1), jnp.float32),
    )(cnt_parts)

    return (loss.reshape(()), q, idx, perp.reshape(()))
